# baseline (speedup 1.0000x reference)
import jax
import jax.numpy as jnp
from jax.experimental import pallas as pl
from jax.experimental.pallas import tpu as pltpu

N, H, W, C = 4, 224, 224, 96
TB = 16


def _upsample_body(prev_ref, mid_ref, next_ref, out_ref):
    for r in range(TB):
        prow = mid_ref[0, r - 1] if r >= 1 else prev_ref[0, 0]
        crow = mid_ref[0, r]
        nrow = mid_ref[0, r + 1] if r < TB - 1 else next_ref[0, 0]
        for a, bl in ((0, 0.25 * prow + 0.75 * crow),
                      (1, 0.75 * crow + 0.25 * nrow)):
            sp = jnp.concatenate([bl[:, :1], bl[:, :-1]], axis=1)
            sn = jnp.concatenate([bl[:, 1:], bl[:, -1:]], axis=1)
            e = 0.25 * sp + 0.75 * bl
            o = 0.75 * bl + 0.25 * sn
            out_ref[0, r, a] = jnp.stack([e, o], axis=2).reshape(C, 2 * W)


def kernel(img):
    imgt = img.transpose(0, 1, 3, 2)
    nblk = H // TB
    out5 = pl.pallas_call(
        _upsample_body,
        grid=(N, nblk),
        in_specs=[
            pl.BlockSpec((1, 1, C, W),
                         lambda n, t: (n, jnp.maximum(t * TB - 1, 0), 0, 0)),
            pl.BlockSpec((1, TB, C, W), lambda n, t: (n, t, 0, 0)),
            pl.BlockSpec((1, 1, C, W),
                         lambda n, t: (n, jnp.minimum(t * TB + TB, H - 1), 0, 0)),
        ],
        out_specs=pl.BlockSpec((1, TB, 2, C, 2 * W),
                               lambda n, t: (n, t, 0, 0, 0)),
        out_shape=jax.ShapeDtypeStruct((N, H, 2, C, 2 * W), img.dtype),
        compiler_params=pltpu.CompilerParams(
            dimension_semantics=("parallel", "arbitrary")),
    )(imgt, imgt, imgt)
    return out5.reshape(N, 2 * H, C, 2 * W).transpose(0, 1, 3, 2)


# speedup vs baseline: 116.6018x; 116.6018x over previous
import numpy as np
import jax
import jax.numpy as jnp
from jax.experimental import pallas as pl
from jax.experimental.pallas import tpu as pltpu

N, H, W, C = 4, 224, 224, 96
TB = 16


def _col_matrix():
    a = np.zeros((W, 2 * W), np.float32)
    for m in range(W):
        a[max(m - 1, 0), 2 * m] += 0.25
        a[m, 2 * m] += 0.75
        a[m, 2 * m + 1] += 0.75
        a[min(m + 1, W - 1), 2 * m + 1] += 0.25
    return a.astype(jnp.bfloat16)


def _upsample_body(prev_ref, mid_ref, next_ref, a_ref, out_ref):
    amat = a_ref[...]
    for r in range(TB):
        prow = mid_ref[0, r - 1] if r >= 1 else prev_ref[0, 0]
        crow = mid_ref[0, r]
        nrow = mid_ref[0, r + 1] if r < TB - 1 else next_ref[0, 0]
        for a, bl in ((0, 0.25 * prow + 0.75 * crow),
                      (1, 0.75 * crow + 0.25 * nrow)):
            out_ref[0, r, a] = jax.lax.dot(
                bl.astype(jnp.bfloat16), amat,
                preferred_element_type=jnp.float32)


def kernel(img):
    imgt = img.transpose(0, 1, 3, 2)
    nblk = H // TB
    out5 = pl.pallas_call(
        _upsample_body,
        grid=(N, nblk),
        in_specs=[
            pl.BlockSpec((1, 1, C, W),
                         lambda n, t: (n, jnp.maximum(t * TB - 1, 0), 0, 0)),
            pl.BlockSpec((1, TB, C, W), lambda n, t: (n, t, 0, 0)),
            pl.BlockSpec((1, 1, C, W),
                         lambda n, t: (n, jnp.minimum(t * TB + TB, H - 1), 0, 0)),
            pl.BlockSpec((W, 2 * W), lambda n, t: (0, 0)),
        ],
        out_specs=pl.BlockSpec((1, TB, 2, C, 2 * W),
                               lambda n, t: (n, t, 0, 0, 0)),
        out_shape=jax.ShapeDtypeStruct((N, H, 2, C, 2 * W), img.dtype),
        compiler_params=pltpu.CompilerParams(
            dimension_semantics=("parallel", "arbitrary")),
    )(imgt, imgt, imgt, _col_matrix())
    return out5.reshape(N, 2 * H, C, 2 * W).transpose(0, 1, 3, 2)
